# baseline (speedup 1.0000x reference)
import functools

import jax
import jax.numpy as jnp
from jax import lax
from jax.experimental import pallas as pl
from jax.experimental.pallas import tpu as pltpu
from jax.experimental.pallas import tpu_sc as plsc

NC, NS, L = 2, 16, 16
NW = NC * NS
NCOL = 26
BATCH = 16384
DIM = 64
SEG = 100000
FLAT = BATCH * NCOL
PER_W = FLAT // NW
CH = 832
NCHUNK = PER_W // CH


def _build():
    mesh = plsc.VectorSubcoreMesh(
        core_axis_name="c", subcore_axis_name="s",
        num_cores=NC, num_subcores=NS,
    )

    @functools.partial(
        pl.kernel,
        out_type=jax.ShapeDtypeStruct((FLAT, DIM), jnp.float32),
        mesh=mesh,
        compiler_params=pltpu.CompilerParams(use_tc_tiling_on_sc=False),
        scratch_types=[
            pltpu.VMEM((PER_W,), jnp.int32),
            pltpu.VMEM((2, CH, DIM), jnp.float32),
            pltpu.SemaphoreType.DMA((2,)),
            pltpu.SemaphoreType.DMA((2,)),
        ],
    )
    def k(x_hbm, table_hbm, out_hbm, idx_v, rows, gsem, ssem):
        wid = lax.axis_index("c") * NS + lax.axis_index("s")
        base = wid * PER_W

        pltpu.sync_copy(x_hbm.at[wid], idx_v)

        lane = jax.lax.iota(jnp.int32, L)

        @pl.loop(0, PER_W // L)
        def _add_offsets(j):
            sl = pl.ds(j * L, L)
            pos = lane + j * L
            idx_v[sl] = idx_v[sl] + (pos % NCOL) * SEG

        def gather(j, b):
            return pltpu.make_async_copy(
                table_hbm.at[idx_v.at[pl.ds(j * CH, CH)]], rows.at[b],
                gsem.at[b])

        def scatter(j, b):
            return pltpu.make_async_copy(
                rows.at[b], out_hbm.at[pl.ds(base + j * CH, CH)], ssem.at[b])

        gather(0, 0).start()
        for j in range(NCHUNK):
            b = j & 1
            gather(j, b).wait()
            if j + 1 < NCHUNK:
                if j >= 1:
                    scatter(j - 1, 1 - b).wait()
                gather(j + 1, 1 - b).start()
            scatter(j, b).start()
        scatter(NCHUNK - 2, (NCHUNK - 2) & 1).wait()
        scatter(NCHUNK - 1, (NCHUNK - 1) & 1).wait()

    return k


_lookup = _build()


def kernel(x_categ, table):
    x_flat = x_categ.astype(jnp.int32).reshape(NW, PER_W)
    out = _lookup(x_flat, table)
    return out.reshape(BATCH, NCOL, DIM)


# speedup vs baseline: 1.0196x; 1.0196x over previous
import functools

import jax
import jax.numpy as jnp
from jax import lax
from jax.experimental import pallas as pl
from jax.experimental.pallas import tpu as pltpu
from jax.experimental.pallas import tpu_sc as plsc

NC, NS, L = 2, 16, 16
NW = NC * NS
NCOL = 26
BATCH = 16384
LOGB = 14
DIM = 64
SEG = 100000
FLAT = BATCH * NCOL
PER_W = FLAT // NW
CH = 832
NCHUNK = PER_W // CH


def _build():
    mesh = plsc.VectorSubcoreMesh(
        core_axis_name="c", subcore_axis_name="s",
        num_cores=NC, num_subcores=NS,
    )

    @functools.partial(
        pl.kernel,
        out_type=jax.ShapeDtypeStruct((FLAT, DIM), jnp.float32),
        mesh=mesh,
        compiler_params=pltpu.CompilerParams(use_tc_tiling_on_sc=False),
        scratch_types=[
            pltpu.VMEM((PER_W,), jnp.int32),
            pltpu.VMEM((2, CH, DIM), jnp.float32),
            pltpu.SemaphoreType.DMA((2,)),
            pltpu.SemaphoreType.DMA((2,)),
        ],
    )
    def k(x_hbm, table_hbm, out_hbm, idx_v, rows, gsem, ssem):
        wid = lax.axis_index("c") * NS + lax.axis_index("s")
        base = wid * PER_W

        pltpu.sync_copy(x_hbm.at[pl.ds(base, PER_W)], idx_v)

        lane = jax.lax.iota(jnp.int32, L)

        @pl.loop(0, PER_W // L)
        def _add_offsets(j):
            sl = pl.ds(j * L, L)
            pos = lane + (base + j * L)
            idx_v[sl] = idx_v[sl] + (pos >> LOGB) * SEG

        def gather(j, b):
            return pltpu.make_async_copy(
                table_hbm.at[idx_v.at[pl.ds(j * CH, CH)]], rows.at[b],
                gsem.at[b])

        def scatter(j, b):
            return pltpu.make_async_copy(
                rows.at[b], out_hbm.at[pl.ds(base + j * CH, CH)], ssem.at[b])

        gather(0, 0).start()
        for j in range(NCHUNK):
            b = j & 1
            gather(j, b).wait()
            if j + 1 < NCHUNK:
                if j >= 1:
                    scatter(j - 1, 1 - b).wait()
                gather(j + 1, 1 - b).start()
            scatter(j, b).start()
        scatter(NCHUNK - 2, (NCHUNK - 2) & 1).wait()
        scatter(NCHUNK - 1, (NCHUNK - 1) & 1).wait()

    return k


_lookup = _build()


def kernel(x_categ, table):
    x_cm = x_categ.astype(jnp.int32).T.reshape(FLAT)
    out = _lookup(x_cm, table)
    return out.reshape(NCOL, BATCH, DIM).transpose(1, 0, 2)
